# baseline (speedup 1.0000x reference)
import functools

import jax
import jax.numpy as jnp
from jax import lax
from jax.experimental import pallas as pl
from jax.experimental.pallas import tpu as pltpu
from jax.experimental.pallas import tpu_sc as plsc

EMB_DIM = 128
N_POS = 128 * 370
NEGATIVE_SIZE = 5
N_NEG = N_POS * NEGATIVE_SIZE

SC_NEG = 75776
TC_NEG = N_NEG - SC_NEG
GRID = 37
POS_BLOCK = N_POS // GRID
NEG_BLOCK = TC_NEG // GRID

NW = 32
SC_R = SC_NEG // NW
SC_C = 32
SC_NCHUNK = SC_R // SC_C
SC_BASE = TC_NEG

_COEF = (6.93186578e-01, 1.24800880e-01, -5.03562042e-03, 2.85647393e-04,
         -1.41225172e-05, 5.10029689e-07, -1.20037919e-08, 1.61785882e-10,
         -9.40153145e-13)


def _tc_loss_kernel(pu_ref, pv_ref, nu_ref, nv_ref, out_ref):
    step = pl.program_id(0)

    def body(u, v, sign):
        n = u.shape[0]
        prod = (u * v).reshape(n // 128, 128, EMB_DIM)
        prod_t = jnp.swapaxes(prod, 1, 2)
        score = jnp.sum(prod_t, axis=1)
        score = jnp.clip(score, -6.0, 6.0)
        return jnp.sum(jnp.log1p(jnp.exp(sign * score)))

    partial = (body(pu_ref[...], pv_ref[...], -1.0)
               + body(nu_ref[...], nv_ref[...], 1.0))

    @pl.when(step == 0)
    def _init():
        out_ref[0, 0] = partial

    @pl.when(step != 0)
    def _acc():
        out_ref[0, 0] += partial


def _tc_call(emb_pos_u, emb_pos_v, emb_neg_u, emb_neg_v):
    pos_spec = pl.BlockSpec((POS_BLOCK, EMB_DIM), lambda i: (i, 0))
    neg_spec = pl.BlockSpec((NEG_BLOCK, EMB_DIM), lambda i: (i, 0))
    return pl.pallas_call(
        _tc_loss_kernel,
        grid=(GRID,),
        in_specs=[pos_spec, pos_spec, neg_spec, neg_spec],
        out_specs=pl.BlockSpec((1, 1), lambda i: (0, 0),
                               memory_space=pltpu.SMEM),
        out_shape=jax.ShapeDtypeStruct((1, 1), jnp.float32),
    )(emb_pos_u, emb_pos_v, emb_neg_u, emb_neg_v)


def _sc_kernel_body(nu_hbm, nv_hbm, out_hbm,
                    ub, vb, accv, us0, us1, vs0, vs1):
    cid = lax.axis_index("c")
    sid = lax.axis_index("s")
    wid = sid * 2 + cid
    base = SC_BASE + wid * SC_R

    usems = (us0, us1)
    vsems = (vs0, vs1)

    def start(i, b):
        row = base + i * SC_C
        pltpu.make_async_copy(nu_hbm.at[pl.ds(row, SC_C)], ub.at[b],
                              usems[b]).start()
        pltpu.make_async_copy(nv_hbm.at[pl.ds(row, SC_C)], vb.at[b],
                              vsems[b]).start()

    def wait(b):
        pltpu.make_async_copy(nu_hbm.at[pl.ds(base, SC_C)], ub.at[b],
                              usems[b]).wait()
        pltpu.make_async_copy(nv_hbm.at[pl.ds(base, SC_C)], vb.at[b],
                              vsems[b]).wait()

    def compute(b, acc):
        for g in range(SC_C // 16):
            rows = lax.iota(jnp.int32, 16) + jnp.full((16,), 16 * g,
                                                      jnp.int32)
            dots = jnp.zeros((16,), jnp.float32)
            for k in range(EMB_DIM):
                cols = jnp.full((16,), k, jnp.int32)
                uvec = plsc.load_gather(ub.at[b], [rows, cols])
                vvec = plsc.load_gather(vb.at[b], [rows, cols])
                dots = dots + uvec * vvec
            s = jnp.clip(dots, -6.0, 6.0)
            y = s * s
            p = jnp.full((16,), _COEF[8], jnp.float32)
            for c in _COEF[7::-1]:
                p = p * y + jnp.full((16,), c, jnp.float32)
            acc = acc + (0.5 * s + p)
        return acc

    start(0, 0)
    start(1, 1)

    def body2(j, acc):
        for b in (0, 1):
            i = j * 2 + b
            wait(b)
            acc = compute(b, acc)

            @pl.when(i + 2 < SC_NCHUNK)
            def _():
                start(i + 2, b)
        return acc

    acc = lax.fori_loop(0, SC_NCHUNK // 2, body2, jnp.zeros((16,),
                                                            jnp.float32))
    accv[...] = acc
    pltpu.sync_copy(accv, out_hbm.at[wid])


def _sc_call(emb_neg_u, emb_neg_v):
    mesh = plsc.VectorSubcoreMesh(core_axis_name="c", subcore_axis_name="s",
                                  num_cores=2, num_subcores=16)
    return pl.kernel(
        _sc_kernel_body,
        out_type=jax.ShapeDtypeStruct((NW, 16), jnp.float32),
        mesh=mesh,
        scratch_types=[
            pltpu.VMEM((2, SC_C, EMB_DIM), jnp.float32),
            pltpu.VMEM((2, SC_C, EMB_DIM), jnp.float32),
            pltpu.VMEM((16,), jnp.float32),
            pltpu.SemaphoreType.DMA,
            pltpu.SemaphoreType.DMA,
            pltpu.SemaphoreType.DMA,
            pltpu.SemaphoreType.DMA,
        ],
        compiler_params=pltpu.CompilerParams(needs_layout_passes=False),
    )(emb_neg_u, emb_neg_v)


def kernel(emb_pos_u, emb_pos_v, emb_neg_u, emb_neg_v):
    sc_out = _sc_call(emb_neg_u, emb_neg_v)
    tc_tot = _tc_call(emb_pos_u, emb_pos_v, emb_neg_u, emb_neg_v)
    return (tc_tot[0, 0] + jnp.sum(sc_out)) / jnp.float32(N_POS)


# speedup vs baseline: 1.2730x; 1.2730x over previous
import functools

import jax
import jax.numpy as jnp
from jax import lax
from jax.experimental import pallas as pl
from jax.experimental.pallas import tpu as pltpu
from jax.experimental.pallas import tpu_sc as plsc

EMB_DIM = 128
N_POS = 128 * 370
NEGATIVE_SIZE = 5
N_NEG = N_POS * NEGATIVE_SIZE

SC_NEG = 75776
TC_NEG = N_NEG - SC_NEG
GRID = 37
POS_BLOCK = N_POS // GRID
NEG_BLOCK = TC_NEG // GRID

NW = 32
SC_R = SC_NEG // NW
SC_C = 32
SC_NCHUNK = SC_R // SC_C
SC_BASE = TC_NEG

_COEF = (6.93186578e-01, 1.24800880e-01, -5.03562042e-03, 2.85647393e-04,
         -1.41225172e-05, 5.10029689e-07, -1.20037919e-08, 1.61785882e-10,
         -9.40153145e-13)


def _tc_loss_kernel(pu_ref, pv_ref, nu_ref, nv_ref, out_ref):
    step = pl.program_id(0)

    def body(u, v, sign):
        n = u.shape[0]
        prod = (u * v).reshape(n // 128, 128, EMB_DIM)
        prod_t = jnp.swapaxes(prod, 1, 2)
        score = jnp.sum(prod_t, axis=1)
        score = jnp.clip(score, -6.0, 6.0)
        return jnp.sum(jnp.log1p(jnp.exp(sign * score)))

    partial = (body(pu_ref[...], pv_ref[...], -1.0)
               + body(nu_ref[...], nv_ref[...], 1.0))

    @pl.when(step == 0)
    def _init():
        out_ref[0, 0] = partial

    @pl.when(step != 0)
    def _acc():
        out_ref[0, 0] += partial


def _tc_call(emb_pos_u, emb_pos_v, emb_neg_u, emb_neg_v):
    pos_spec = pl.BlockSpec((POS_BLOCK, EMB_DIM), lambda i: (i, 0))
    neg_spec = pl.BlockSpec((NEG_BLOCK, EMB_DIM), lambda i: (i, 0))
    return pl.pallas_call(
        _tc_loss_kernel,
        grid=(GRID,),
        in_specs=[pos_spec, pos_spec, neg_spec, neg_spec],
        out_specs=pl.BlockSpec((1, 1), lambda i: (0, 0),
                               memory_space=pltpu.SMEM),
        out_shape=jax.ShapeDtypeStruct((1, 1), jnp.float32),
    )(emb_pos_u, emb_pos_v, emb_neg_u, emb_neg_v)


def _sc_kernel_body(nu_hbm, nv_hbm, out_hbm,
                    ub0, ub1, vb0, vb1, accv, us0, us1, vs0, vs1):
    ubufs = (ub0, ub1)
    vbufs = (vb0, vb1)
    cid = lax.axis_index("c")
    sid = lax.axis_index("s")
    wid = sid * 2 + cid
    base = (SC_BASE + wid * SC_R) * EMB_DIM

    usems = (us0, us1)
    vsems = (vs0, vs1)
    chunk_f32 = SC_C * EMB_DIM

    def start(i, b):
        off = base + i * chunk_f32
        pltpu.make_async_copy(nu_hbm.at[pl.ds(off, chunk_f32)], ubufs[b],
                              usems[b]).start()
        pltpu.make_async_copy(nv_hbm.at[pl.ds(off, chunk_f32)], vbufs[b],
                              vsems[b]).start()

    def wait(b):
        pltpu.make_async_copy(nu_hbm.at[pl.ds(base, chunk_f32)], ubufs[b],
                              usems[b]).wait()
        pltpu.make_async_copy(nv_hbm.at[pl.ds(base, chunk_f32)], vbufs[b],
                              vsems[b]).wait()

    lane128 = lax.iota(jnp.int32, 16) * jnp.full((16,), EMB_DIM, jnp.int32)
    one = jnp.full((16,), 1, jnp.int32)

    def compute(b, acc):
        for g in range(SC_C // 16):
            idx0 = lane128 + jnp.full((16,), g * 16 * EMB_DIM, jnp.int32)
            zero = jnp.zeros((16,), jnp.float32)

            def kbody(t, carry):
                idx = carry[0]
                accs = list(carry[1:])
                for kk in range(16):
                    uvec = plsc.load_gather(ubufs[b], [idx])
                    vvec = plsc.load_gather(vbufs[b], [idx])
                    idx = idx + one
                    accs[kk % 8] = accs[kk % 8] + uvec * vvec
                return (idx, *accs)

            carry = lax.fori_loop(0, EMB_DIM // 16, kbody,
                                  (idx0,) + (zero,) * 8)
            accs = carry[1:]
            dots = ((accs[0] + accs[1]) + (accs[2] + accs[3])) + (
                (accs[4] + accs[5]) + (accs[6] + accs[7]))
            s = jnp.clip(dots, -6.0, 6.0)
            y = s * s
            p = jnp.full((16,), _COEF[8], jnp.float32)
            for c in _COEF[7::-1]:
                p = p * y + jnp.full((16,), c, jnp.float32)
            acc = acc + (0.5 * s + p)
        return acc

    start(0, 0)
    start(1, 1)

    def body2(j, acc):
        for b in (0, 1):
            i = j * 2 + b
            wait(b)
            acc = compute(b, acc)

            @pl.when(i + 2 < SC_NCHUNK)
            def _():
                start(i + 2, b)
        return acc

    acc = lax.fori_loop(0, SC_NCHUNK // 2, body2, jnp.zeros((16,),
                                                            jnp.float32))
    accv[...] = acc
    pltpu.sync_copy(accv, out_hbm.at[wid])


def _sc_call(emb_neg_u, emb_neg_v):
    mesh = plsc.VectorSubcoreMesh(core_axis_name="c", subcore_axis_name="s",
                                  num_cores=2, num_subcores=16)
    return pl.kernel(
        _sc_kernel_body,
        out_type=jax.ShapeDtypeStruct((NW, 16), jnp.float32),
        mesh=mesh,
        scratch_types=[
            pltpu.VMEM((SC_C * EMB_DIM,), jnp.float32),
            pltpu.VMEM((SC_C * EMB_DIM,), jnp.float32),
            pltpu.VMEM((SC_C * EMB_DIM,), jnp.float32),
            pltpu.VMEM((SC_C * EMB_DIM,), jnp.float32),
            pltpu.VMEM((16,), jnp.float32),
            pltpu.SemaphoreType.DMA,
            pltpu.SemaphoreType.DMA,
            pltpu.SemaphoreType.DMA,
            pltpu.SemaphoreType.DMA,
        ],
        compiler_params=pltpu.CompilerParams(needs_layout_passes=False),
    )(emb_neg_u, emb_neg_v)


def kernel(emb_pos_u, emb_pos_v, emb_neg_u, emb_neg_v):
    sc_out = _sc_call(emb_neg_u.reshape(-1), emb_neg_v.reshape(-1))
    tc_tot = _tc_call(emb_pos_u, emb_pos_v, emb_neg_u, emb_neg_v)
    return (tc_tot[0, 0] + jnp.sum(sc_out)) / jnp.float32(N_POS)
